# baseline (speedup 1.0000x reference)
import jax
import jax.numpy as jnp
from jax import lax
from jax.experimental import pallas as pl
from jax.experimental.pallas import tpu as pltpu
from jax.experimental.pallas import tpu_sc as plsc

HEIGHT = 32
WIDTH = 32
CH = 192
B = 64
HW = HEIGHT * WIDTH
L = 16

NC = 2
NS = 16
NCG = 8
NBG = 4
CPW = CH // NCG
BPW = B // NBG

CB = 2
PW = 256
NBUF = 4
NPC = HW // PW
NCHUNK = (BPW // CB) * NPC
PF = 2


def _sc_body(x_hbm, h_hbm, w_hbm, out_hbm,
             h_v, w_v, pos_v, buf0, buf1, buf2, buf3,
             in_s0, in_s1, in_s2, in_s3, out_s0, out_s1, out_s2, out_s3):
    bufs = (buf0, buf1, buf2, buf3)
    in_sems = (in_s0, in_s1, in_s2, in_s3)
    out_sems = (out_s0, out_s1, out_s2, out_s3)

    wid = lax.axis_index("s") * NC + lax.axis_index("c")
    cg = lax.rem(wid, NCG)
    bg = lax.div(wid, NCG)
    c0 = cg * CPW
    b0 = bg * BPW

    pltpu.sync_copy(h_hbm, h_v)
    pltpu.sync_copy(w_hbm, w_v)

    lanes = lax.iota(jnp.int32, L)

    def posq(q, carry):
        jj = q // HEIGHT
        h = q % HEIGHT
        cidx = jnp.full((L,), c0 + jj, jnp.int32)
        hvv = plsc.load_gather(h_v, [jnp.full((L,), h, jnp.int32), cidx])
        wv0 = plsc.load_gather(w_v, [lanes, cidx])
        wv1 = plsc.load_gather(w_v, [lanes + L, cidx])
        pos_v[jj, pl.ds(h * WIDTH, L)] = hvv + wv0
        pos_v[jj, pl.ds(h * WIDTH + L, L)] = hvv + wv1
        return carry

    lax.fori_loop(0, CPW * HEIGHT, posq, 0)

    def chunk_coords(idx):
        return b0 + (idx // NPC) * CB, (idx % NPC) * PW

    def in_slice(idx):
        cb, cp = chunk_coords(idx)
        return x_hbm.at[pl.ds(cb, CB), pl.ds(c0, CPW), pl.ds(cp, PW)]

    def out_slice(idx):
        cb, cp = chunk_coords(idx)
        return out_hbm.at[pl.ds(cb, CB), pl.ds(c0, CPW), pl.ds(cp, PW)]

    for t in range(NBUF):
        pltpu.async_copy(in_slice(t), bufs[t], in_sems[t])

    def round_body(g, carry):
        for t in range(NBUF):
            idx = g * NBUF + t
            buf = bufs[t]
            pltpu.make_async_copy(in_slice(idx), buf, in_sems[t]).wait()

            _, cp = chunk_coords(idx)

            def ck(k, kcarry, buf=buf, cp=cp):
                for j in range(CPW):
                    pv = pos_v[j, pl.ds(cp + k * L, L)]
                    for b in range(CB):
                        plsc.addupdate(buf.at[b, j, pl.ds(k * L, L)], pv)
                return kcarry

            lax.fori_loop(0, PW // L, ck, 0)

            pltpu.async_copy(buf, out_slice(idx), out_sems[t])

            nxt = idx + PF
            t2 = (t + PF) % NBUF

            @pl.when(jnp.logical_and(nxt >= NBUF, nxt < NCHUNK))
            def _refill(nxt=nxt, t2=t2):
                pltpu.make_async_copy(bufs[t2], out_slice(nxt - NBUF),
                                      out_sems[t2]).wait()
                pltpu.async_copy(in_slice(nxt), bufs[t2], in_sems[t2])

        return carry

    lax.fori_loop(0, NCHUNK // NBUF, round_body, 0)

    for idx in range(NCHUNK - NBUF, NCHUNK):
        t = idx % NBUF
        pltpu.make_async_copy(bufs[t], out_slice(idx), out_sems[t]).wait()


def kernel(x, h_emb, w_emb):
    b, c, h, w = x.shape
    xf = x.reshape(b, c, h * w)

    mesh = plsc.VectorSubcoreMesh(core_axis_name="c", subcore_axis_name="s")
    run = pl.kernel(
        _sc_body,
        mesh=mesh,
        compiler_params=pltpu.CompilerParams(needs_layout_passes=False),
        out_type=jax.ShapeDtypeStruct((b, c, h * w), jnp.float32),
        scratch_types=[
            pltpu.VMEM((HEIGHT, CH), jnp.float32),
            pltpu.VMEM((WIDTH, CH), jnp.float32),
            pltpu.VMEM((CPW, HW), jnp.float32),
            pltpu.VMEM((CB, CPW, PW), jnp.float32),
            pltpu.VMEM((CB, CPW, PW), jnp.float32),
            pltpu.VMEM((CB, CPW, PW), jnp.float32),
            pltpu.VMEM((CB, CPW, PW), jnp.float32),
            pltpu.SemaphoreType.DMA,
            pltpu.SemaphoreType.DMA,
            pltpu.SemaphoreType.DMA,
            pltpu.SemaphoreType.DMA,
            pltpu.SemaphoreType.DMA,
            pltpu.SemaphoreType.DMA,
            pltpu.SemaphoreType.DMA,
            pltpu.SemaphoreType.DMA,
        ],
    )
    out = run(xf, h_emb, w_emb)
    return out.reshape(b, c, h, w)
